# baseline (speedup 1.0000x reference)
import jax
import jax.numpy as jnp
from jax import lax
from jax.experimental import pallas as pl
from jax.experimental.pallas import tpu as pltpu
from jax.experimental.pallas import tpu_sc as plsc

B, S, L = 4, 8192, 4096

SC_ROWS = 2048
TC_ROWS = S - SC_ROWS

NC, NSUB = 2, 16
NW = NC * NSUB
WROWS = SC_ROWS // NW
RCH = 8
NCH = WROWS // RCH
NG = L // 16

SBLK = 1024
NSB = TC_ROWS // SBLK


def _sc_body(x_hbm, pv_hbm, pi_hbm, buf0, buf1, sval, sidx, sem0, sem1):
    wid = lax.axis_index("c") * NSUB + lax.axis_index("s")
    r0 = TC_ROWS + wid * WROWS

    def copy_in(b, ch, buf, sem):
        return pltpu.make_async_copy(
            x_hbm.at[b, pl.ds(r0 + ch * RCH, RCH), :], buf, sem)

    def chunk_compute(buf, row_base):
        rvecs = [jnp.full((16,), row_base + r, dtype=jnp.int32)
                 for r in range(RCH)]

        def group_body(g, _):
            off = g * 16
            sv = sval[pl.ds(off, 16)]
            si = sidx[pl.ds(off, 16)]
            for r in range(RCH):
                v = buf[r, pl.ds(off, 16)]
                m = v < sv
                sv = jnp.where(m, v, sv)
                si = jnp.where(m, rvecs[r], si)
            sval[pl.ds(off, 16)] = sv
            sidx[pl.ds(off, 16)] = si
            return 0

        lax.fori_loop(0, NG, group_body, 0, unroll=2)

    inf16 = jnp.full((16,), jnp.inf, dtype=jnp.float32)
    zero16 = jnp.zeros((16,), dtype=jnp.int32)

    for b in range(B):
        copy_in(b, 0, buf0, sem0).start()
        copy_in(b, 1, buf1, sem1).start()

        def init_body(g, _):
            sval[pl.ds(g * 16, 16)] = inf16
            sidx[pl.ds(g * 16, 16)] = zero16
            return 0

        lax.fori_loop(0, NG, init_body, 0, unroll=4)

        def pair_body(p, _, b=b):
            copy_in(b, 2 * p, buf0, sem0).wait()
            chunk_compute(buf0, r0 + 2 * p * RCH)

            @pl.when(p + 1 < NCH // 2)
            def _():
                copy_in(b, 2 * p + 2, buf0, sem0).start()

            copy_in(b, 2 * p + 1, buf1, sem1).wait()
            chunk_compute(buf1, r0 + (2 * p + 1) * RCH)

            @pl.when(p + 1 < NCH // 2)
            def _():
                copy_in(b, 2 * p + 3, buf1, sem1).start()

            return 0

        lax.fori_loop(0, NCH // 2, pair_body, 0)
        pltpu.sync_copy(sval, pv_hbm.at[b, pl.ds(wid * L, L)])
        pltpu.sync_copy(sidx, pi_hbm.at[b, pl.ds(wid * L, L)])


def _sc_argmin(x):
    mesh = plsc.VectorSubcoreMesh(core_axis_name="c", subcore_axis_name="s")
    return pl.kernel(
        _sc_body,
        out_type=(
            jax.ShapeDtypeStruct((B, NW * L), jnp.float32),
            jax.ShapeDtypeStruct((B, NW * L), jnp.int32),
        ),
        mesh=mesh,
        scratch_types=[
            pltpu.VMEM((RCH, L), jnp.float32),
            pltpu.VMEM((RCH, L), jnp.float32),
            pltpu.VMEM((L,), jnp.float32),
            pltpu.VMEM((L,), jnp.int32),
            pltpu.SemaphoreType.DMA,
            pltpu.SemaphoreType.DMA,
        ],
    )(x)


def _tc_body(x_ref, ov_ref, oi_ref, mv_ref, mi_ref):
    s = pl.program_id(1)
    v = x_ref[0]
    m = jnp.min(v, axis=0)
    iota = jax.lax.broadcasted_iota(jnp.int32, v.shape, 0)
    idx = jnp.min(jnp.where(v == m[None, :], iota, jnp.int32(S)), axis=0) + s * SBLK

    @pl.when(s == 0)
    def _():
        mv_ref[0] = m
        mi_ref[0] = idx

    @pl.when(s > 0)
    def _():
        better = m < mv_ref[0]
        mi_ref[0] = jnp.where(better, idx, mi_ref[0])
        mv_ref[0] = jnp.where(better, m, mv_ref[0])

    @pl.when(s == NSB - 1)
    def _():
        ov_ref[0] = mv_ref[...]
        oi_ref[0] = mi_ref[...]


def _tc_argmin(x):
    ov, oi = pl.pallas_call(
        _tc_body,
        grid=(B, NSB),
        in_specs=[pl.BlockSpec((1, SBLK, L), lambda b, s: (b, s, 0))],
        out_specs=[
            pl.BlockSpec((1, 1, L), lambda b, s: (b, 0, 0)),
            pl.BlockSpec((1, 1, L), lambda b, s: (b, 0, 0)),
        ],
        out_shape=[
            jax.ShapeDtypeStruct((B, 1, L), jnp.float32),
            jax.ShapeDtypeStruct((B, 1, L), jnp.int32),
        ],
        scratch_shapes=[
            pltpu.VMEM((1, L), jnp.float32),
            pltpu.VMEM((1, L), jnp.int32),
        ],
    )(x)
    return ov, oi


def _merge_body(tv_ref, ti_ref, pv_ref, pi_ref, o_ref):
    pv = pv_ref[0]
    pi = pi_ref[0]
    m_sc = jnp.min(pv, axis=0)
    i_sc = jnp.min(jnp.where(pv == m_sc[None, :], pi, jnp.int32(S)), axis=0)
    tv = tv_ref[0, 0]
    ti = ti_ref[0, 0]
    o_ref[0, 0] = jnp.where(tv <= m_sc, ti, i_sc)


def _merge(tv, ti, pv, pi):
    out = pl.pallas_call(
        _merge_body,
        grid=(B,),
        in_specs=[
            pl.BlockSpec((1, 1, L), lambda b: (b, 0, 0)),
            pl.BlockSpec((1, 1, L), lambda b: (b, 0, 0)),
            pl.BlockSpec((1, NW, L), lambda b: (b, 0, 0)),
            pl.BlockSpec((1, NW, L), lambda b: (b, 0, 0)),
        ],
        out_specs=pl.BlockSpec((1, 1, L), lambda b: (b, 0, 0)),
        out_shape=jax.ShapeDtypeStruct((B, 1, L), jnp.int32),
    )(tv, ti, pv.reshape(B, NW, L), pi.reshape(B, NW, L))
    return out.reshape(B, L)


def kernel(x):
    pv, pi = _sc_argmin(x)
    tv, ti = _tc_argmin(x)
    return _merge(tv, ti, pv, pi)


# speedup vs baseline: 1.0477x; 1.0477x over previous
import jax
import jax.numpy as jnp
from jax import lax
from jax.experimental import pallas as pl
from jax.experimental.pallas import tpu as pltpu
from jax.experimental.pallas import tpu_sc as plsc

B, S, L = 4, 8192, 4096

SC_ROWS = 2048
TC_ROWS = S - SC_ROWS

NC, NSUB = 2, 16
NW = NC * NSUB
WROWS = SC_ROWS // NW
RCH = 8
NCH = WROWS // RCH
NG = L // 16

SBLK = 1024
NSB = TC_ROWS // SBLK


def _sc_body(x_hbm, pv_hbm, pi_hbm, buf0, buf1, sval, sidx, sem0, sem1):
    wid = lax.axis_index("c") * NSUB + lax.axis_index("s")
    r0 = TC_ROWS + wid * WROWS

    def copy_in(b, ch, buf, sem):
        return pltpu.make_async_copy(
            x_hbm.at[b, pl.ds(r0 + ch * RCH, RCH), :], buf, sem)

    def chunk_compute(buf, row_base):
        rvecs = [jnp.full((16,), row_base + r, dtype=jnp.int32)
                 for r in range(RCH)]

        def group_body(g, _):
            off = g * 16
            pairs = [(buf[r, pl.ds(off, 16)], rvecs[r]) for r in range(RCH)]
            while len(pairs) > 1:
                nxt = []
                for k in range(0, len(pairs), 2):
                    (va, ia), (vb, ib) = pairs[k], pairs[k + 1]
                    m = vb < va
                    nxt.append((jnp.where(m, vb, va), jnp.where(m, ib, ia)))
                pairs = nxt
            cv, ci = pairs[0]
            sv = sval[pl.ds(off, 16)]
            si = sidx[pl.ds(off, 16)]
            m = cv < sv
            sval[pl.ds(off, 16)] = jnp.where(m, cv, sv)
            sidx[pl.ds(off, 16)] = jnp.where(m, ci, si)
            return 0

        lax.fori_loop(0, NG, group_body, 0, unroll=4)

    inf16 = jnp.full((16,), jnp.inf, dtype=jnp.float32)
    zero16 = jnp.zeros((16,), dtype=jnp.int32)

    for b in range(B):
        copy_in(b, 0, buf0, sem0).start()
        copy_in(b, 1, buf1, sem1).start()

        def init_body(g, _):
            sval[pl.ds(g * 16, 16)] = inf16
            sidx[pl.ds(g * 16, 16)] = zero16
            return 0

        lax.fori_loop(0, NG, init_body, 0, unroll=4)

        def pair_body(p, _, b=b):
            copy_in(b, 2 * p, buf0, sem0).wait()
            chunk_compute(buf0, r0 + 2 * p * RCH)

            @pl.when(p + 1 < NCH // 2)
            def _():
                copy_in(b, 2 * p + 2, buf0, sem0).start()

            copy_in(b, 2 * p + 1, buf1, sem1).wait()
            chunk_compute(buf1, r0 + (2 * p + 1) * RCH)

            @pl.when(p + 1 < NCH // 2)
            def _():
                copy_in(b, 2 * p + 3, buf1, sem1).start()

            return 0

        lax.fori_loop(0, NCH // 2, pair_body, 0)
        pltpu.sync_copy(sval, pv_hbm.at[b, wid])
        pltpu.sync_copy(sidx, pi_hbm.at[b, wid])


def _sc_argmin(x):
    mesh = plsc.VectorSubcoreMesh(core_axis_name="c", subcore_axis_name="s")
    return pl.kernel(
        _sc_body,
        out_type=(
            jax.ShapeDtypeStruct((B, NW, L), jnp.float32),
            jax.ShapeDtypeStruct((B, NW, L), jnp.int32),
        ),
        mesh=mesh,
        scratch_types=[
            pltpu.VMEM((RCH, L), jnp.float32),
            pltpu.VMEM((RCH, L), jnp.float32),
            pltpu.VMEM((L,), jnp.float32),
            pltpu.VMEM((L,), jnp.int32),
            pltpu.SemaphoreType.DMA,
            pltpu.SemaphoreType.DMA,
        ],
    )(x)


def _tc_body(x_ref, ov_ref, oi_ref, mv_ref, mi_ref):
    s = pl.program_id(1)
    v = x_ref[0]
    m = jnp.min(v, axis=0)
    iota = jax.lax.broadcasted_iota(jnp.int32, v.shape, 0)
    idx = jnp.min(jnp.where(v == m[None, :], iota, jnp.int32(S)), axis=0) + s * SBLK

    @pl.when(s == 0)
    def _():
        mv_ref[0] = m
        mi_ref[0] = idx

    @pl.when(s > 0)
    def _():
        better = m < mv_ref[0]
        mi_ref[0] = jnp.where(better, idx, mi_ref[0])
        mv_ref[0] = jnp.where(better, m, mv_ref[0])

    @pl.when(s == NSB - 1)
    def _():
        ov_ref[0] = mv_ref[...]
        oi_ref[0] = mi_ref[...]


def _tc_argmin(x):
    ov, oi = pl.pallas_call(
        _tc_body,
        grid=(B, NSB),
        in_specs=[pl.BlockSpec((1, SBLK, L), lambda b, s: (b, s, 0))],
        out_specs=[
            pl.BlockSpec((1, 1, L), lambda b, s: (b, 0, 0)),
            pl.BlockSpec((1, 1, L), lambda b, s: (b, 0, 0)),
        ],
        out_shape=[
            jax.ShapeDtypeStruct((B, 1, L), jnp.float32),
            jax.ShapeDtypeStruct((B, 1, L), jnp.int32),
        ],
        scratch_shapes=[
            pltpu.VMEM((1, L), jnp.float32),
            pltpu.VMEM((1, L), jnp.int32),
        ],
    )(x)
    return ov, oi


def _merge_body(tv_ref, ti_ref, pv_ref, pi_ref, o_ref):
    pv = pv_ref[...]
    pi = pi_ref[...]
    m_sc = jnp.min(pv, axis=1)
    i_sc = jnp.min(jnp.where(pv == m_sc[:, None, :], pi, jnp.int32(S)), axis=1)
    tv = tv_ref[:, 0, :]
    ti = ti_ref[:, 0, :]
    o_ref[...] = jnp.where(tv <= m_sc, ti, i_sc)


def _merge(tv, ti, pv, pi):
    return pl.pallas_call(
        _merge_body,
        out_shape=jax.ShapeDtypeStruct((B, L), jnp.int32),
    )(tv, ti, pv, pi)


def kernel(x):
    pv, pi = _sc_argmin(x)
    tv, ti = _tc_argmin(x)
    return _merge(tv, ti, pv, pi)


# speedup vs baseline: 1.0548x; 1.0067x over previous
import jax
import jax.numpy as jnp
from jax import lax
from jax.experimental import pallas as pl
from jax.experimental.pallas import tpu as pltpu
from jax.experimental.pallas import tpu_sc as plsc

B, S, L = 4, 8192, 4096

SC_ROWS = 2048
TC_ROWS = S - SC_ROWS

NC, NSUB = 2, 16
NW = NC * NSUB
WROWS = SC_ROWS // NW
RCH = 8
NCH = WROWS // RCH
NG = L // 16

SBLK = 1024
NSB = TC_ROWS // SBLK


def _sc_body(x_hbm, pv_hbm, pi_hbm, buf0, buf1, sval, sidx, sem0, sem1):
    wid = lax.axis_index("c") * NSUB + lax.axis_index("s")
    r0 = TC_ROWS + wid * WROWS

    def copy_in(b, ch, buf, sem):
        return pltpu.make_async_copy(
            x_hbm.at[b, pl.ds(r0 + ch * RCH, RCH), :], buf, sem)

    def chunk_compute(buf, row_base):
        rvecs = [jnp.full((16,), row_base + r, dtype=jnp.int32)
                 for r in range(RCH)]

        def group_body(g, _):
            off = g * 16
            pairs = [(buf[r, pl.ds(off, 16)], rvecs[r]) for r in range(RCH)]
            while len(pairs) > 1:
                nxt = []
                for k in range(0, len(pairs), 2):
                    (va, ia), (vb, ib) = pairs[k], pairs[k + 1]
                    m = vb < va
                    nxt.append((jnp.where(m, vb, va), jnp.where(m, ib, ia)))
                pairs = nxt
            cv, ci = pairs[0]
            sv = sval[pl.ds(off, 16)]
            si = sidx[pl.ds(off, 16)]
            m = cv < sv
            sval[pl.ds(off, 16)] = jnp.where(m, cv, sv)
            sidx[pl.ds(off, 16)] = jnp.where(m, ci, si)
            return 0

        lax.fori_loop(0, NG, group_body, 0, unroll=4)

    inf16 = jnp.full((16,), jnp.inf, dtype=jnp.float32)
    zero16 = jnp.zeros((16,), dtype=jnp.int32)

    for b in range(B):
        copy_in(b, 0, buf0, sem0).start()
        copy_in(b, 1, buf1, sem1).start()

        def init_body(g, _):
            sval[pl.ds(g * 16, 16)] = inf16
            sidx[pl.ds(g * 16, 16)] = zero16
            return 0

        lax.fori_loop(0, NG, init_body, 0, unroll=4)

        def pair_body(p, _, b=b):
            copy_in(b, 2 * p, buf0, sem0).wait()
            chunk_compute(buf0, r0 + 2 * p * RCH)

            @pl.when(p + 1 < NCH // 2)
            def _():
                copy_in(b, 2 * p + 2, buf0, sem0).start()

            copy_in(b, 2 * p + 1, buf1, sem1).wait()
            chunk_compute(buf1, r0 + (2 * p + 1) * RCH)

            @pl.when(p + 1 < NCH // 2)
            def _():
                copy_in(b, 2 * p + 3, buf1, sem1).start()

            return 0

        lax.fori_loop(0, NCH // 2, pair_body, 0)
        pltpu.sync_copy(sval, pv_hbm.at[b, wid])
        pltpu.sync_copy(sidx, pi_hbm.at[b, wid])


def _sc_argmin(x):
    mesh = plsc.VectorSubcoreMesh(core_axis_name="c", subcore_axis_name="s")
    return pl.kernel(
        _sc_body,
        out_type=(
            jax.ShapeDtypeStruct((B, NW, L), jnp.float32),
            jax.ShapeDtypeStruct((B, NW, L), jnp.int32),
        ),
        mesh=mesh,
        scratch_types=[
            pltpu.VMEM((RCH, L), jnp.float32),
            pltpu.VMEM((RCH, L), jnp.float32),
            pltpu.VMEM((L,), jnp.float32),
            pltpu.VMEM((L,), jnp.int32),
            pltpu.SemaphoreType.DMA,
            pltpu.SemaphoreType.DMA,
        ],
    )(x)


LH = L // 2


def _tc_body(xa_ref, xb_ref, ov_ref, oi_ref, mv_ref, mi_ref):
    s = pl.program_id(1)
    for h, x_ref in enumerate((xa_ref, xb_ref)):
        cs = pl.ds(h * LH, LH)
        v = x_ref[0]
        m = jnp.min(v, axis=0)
        iota = jax.lax.broadcasted_iota(jnp.int32, v.shape, 0)
        idx = jnp.min(
            jnp.where(v == m[None, :], iota, jnp.int32(S)), axis=0) + s * SBLK

        @pl.when(s == 0)
        def _(m=m, idx=idx, cs=cs):
            mv_ref[0, cs] = m
            mi_ref[0, cs] = idx

        @pl.when(s > 0)
        def _(m=m, idx=idx, cs=cs):
            better = m < mv_ref[0, cs]
            mi_ref[0, cs] = jnp.where(better, idx, mi_ref[0, cs])
            mv_ref[0, cs] = jnp.where(better, m, mv_ref[0, cs])

    @pl.when(s == NSB - 1)
    def _():
        ov_ref[0] = mv_ref[...]
        oi_ref[0] = mi_ref[...]


def _tc_argmin(x):
    ov, oi = pl.pallas_call(
        _tc_body,
        grid=(B, NSB),
        in_specs=[
            pl.BlockSpec((1, SBLK, LH), lambda b, s: (b, s, 0)),
            pl.BlockSpec((1, SBLK, LH), lambda b, s: (b, s, 1)),
        ],
        out_specs=[
            pl.BlockSpec((1, 1, L), lambda b, s: (b, 0, 0)),
            pl.BlockSpec((1, 1, L), lambda b, s: (b, 0, 0)),
        ],
        out_shape=[
            jax.ShapeDtypeStruct((B, 1, L), jnp.float32),
            jax.ShapeDtypeStruct((B, 1, L), jnp.int32),
        ],
        scratch_shapes=[
            pltpu.VMEM((1, L), jnp.float32),
            pltpu.VMEM((1, L), jnp.int32),
        ],
    )(x, x)
    return ov, oi


def _merge_body(tv_ref, ti_ref, pv_ref, pi_ref, o_ref):
    pv = pv_ref[...]
    pi = pi_ref[...]
    m_sc = jnp.min(pv, axis=1)
    i_sc = jnp.min(jnp.where(pv == m_sc[:, None, :], pi, jnp.int32(S)), axis=1)
    tv = tv_ref[:, 0, :]
    ti = ti_ref[:, 0, :]
    o_ref[...] = jnp.where(tv <= m_sc, ti, i_sc)


def _merge(tv, ti, pv, pi):
    return pl.pallas_call(
        _merge_body,
        out_shape=jax.ShapeDtypeStruct((B, L), jnp.int32),
    )(tv, ti, pv, pi)


def kernel(x):
    pv, pi = _sc_argmin(x)
    tv, ti = _tc_argmin(x)
    return _merge(tv, ti, pv, pi)


# speedup vs baseline: 1.0755x; 1.0196x over previous
import jax
import jax.numpy as jnp
from jax import lax
from jax.experimental import pallas as pl
from jax.experimental.pallas import tpu as pltpu
from jax.experimental.pallas import tpu_sc as plsc

B, S, L = 4, 8192, 4096

SC_COLS = 1024
TC_COLS = L - SC_COLS

NC, NSUB = 2, 16
NW = NC * NSUB
CW = 128
NSTRIPE = SC_COLS // CW
NQ = NW // NSTRIPE
QROWS = S // NQ
G = CW // 16
R = 256
NCH = QROWS // R

SBLK = 1024
NSB = S // SBLK
LH = TC_COLS // 2


def _sc_body(x_hbm, pv_hbm, pi_hbm, buf0, buf1, obv, obi, sem0, sem1):
    wid = lax.axis_index("c") * NSUB + lax.axis_index("s")
    stripe = wid // NQ
    q = wid % NQ
    c0 = TC_COLS + stripe * CW
    r0 = q * QROWS
    oc = q * SC_COLS + stripe * CW

    def copy_in(b, ch, buf, sem):
        return pltpu.make_async_copy(
            x_hbm.at[b, pl.ds(r0 + ch * R, R), pl.ds(c0, CW)], buf, sem)

    def rowloop(buf, base, carry):
        def row_body(r, cr):
            mins, idxs = cr
            rvec = jnp.full((16,), base + r, dtype=jnp.int32)
            nm, ni = [], []
            for g in range(G):
                v = buf[r, pl.ds(g * 16, 16)]
                m = v < mins[g]
                nm.append(jnp.where(m, v, mins[g]))
                ni.append(jnp.where(m, rvec, idxs[g]))
            return (tuple(nm), tuple(ni))
        return lax.fori_loop(0, R, row_body, carry, unroll=4)

    for b in range(B):
        copy_in(b, 0, buf0, sem0).start()
        copy_in(b, 1, buf1, sem1).start()
        init = (
            tuple(jnp.full((16,), jnp.inf, jnp.float32) for _ in range(G)),
            tuple(jnp.zeros((16,), jnp.int32) for _ in range(G)),
        )

        def pair_body(p, carry, b=b):
            copy_in(b, 2 * p, buf0, sem0).wait()
            carry = rowloop(buf0, r0 + 2 * p * R, carry)

            @pl.when(p + 1 < NCH // 2)
            def _():
                copy_in(b, 2 * p + 2, buf0, sem0).start()

            copy_in(b, 2 * p + 1, buf1, sem1).wait()
            carry = rowloop(buf1, r0 + (2 * p + 1) * R, carry)

            @pl.when(p + 1 < NCH // 2)
            def _():
                copy_in(b, 2 * p + 3, buf1, sem1).start()

            return carry

        mins, idxs = lax.fori_loop(0, NCH // 2, pair_body, init)
        for g in range(G):
            obv[pl.ds(g * 16, 16)] = mins[g]
            obi[pl.ds(g * 16, 16)] = idxs[g]
        pltpu.sync_copy(obv, pv_hbm.at[b, pl.ds(oc, CW)])
        pltpu.sync_copy(obi, pi_hbm.at[b, pl.ds(oc, CW)])


def _sc_argmin(x):
    mesh = plsc.VectorSubcoreMesh(core_axis_name="c", subcore_axis_name="s")
    return pl.kernel(
        _sc_body,
        out_type=(
            jax.ShapeDtypeStruct((B, NQ * SC_COLS), jnp.float32),
            jax.ShapeDtypeStruct((B, NQ * SC_COLS), jnp.int32),
        ),
        mesh=mesh,
        scratch_types=[
            pltpu.VMEM((R, CW), jnp.float32),
            pltpu.VMEM((R, CW), jnp.float32),
            pltpu.VMEM((CW,), jnp.float32),
            pltpu.VMEM((CW,), jnp.int32),
            pltpu.SemaphoreType.DMA,
            pltpu.SemaphoreType.DMA,
        ],
    )(x)


def _tc_body(xa_ref, xb_ref, o_ref, mv_ref, mi_ref):
    s = pl.program_id(1)
    for h, x_ref in enumerate((xa_ref, xb_ref)):
        cs = pl.ds(h * LH, LH)
        v = x_ref[0]
        m = jnp.min(v, axis=0)
        iota = jax.lax.broadcasted_iota(jnp.int32, v.shape, 0)
        idx = jnp.min(
            jnp.where(v == m[None, :], iota, jnp.int32(S)), axis=0) + s * SBLK

        @pl.when(s == 0)
        def _(m=m, idx=idx, cs=cs):
            mv_ref[0, cs] = m
            mi_ref[0, cs] = idx

        @pl.when(s > 0)
        def _(m=m, idx=idx, cs=cs):
            better = m < mv_ref[0, cs]
            mi_ref[0, cs] = jnp.where(better, idx, mi_ref[0, cs])
            mv_ref[0, cs] = jnp.where(better, m, mv_ref[0, cs])

    @pl.when(s == NSB - 1)
    def _():
        o_ref[0] = mi_ref[...]


def _tc_argmin(x):
    return pl.pallas_call(
        _tc_body,
        grid=(B, NSB),
        in_specs=[
            pl.BlockSpec((1, SBLK, LH), lambda b, s: (b, s, 0)),
            pl.BlockSpec((1, SBLK, LH), lambda b, s: (b, s, 1)),
        ],
        out_specs=pl.BlockSpec((1, 1, TC_COLS), lambda b, s: (b, 0, 0)),
        out_shape=jax.ShapeDtypeStruct((B, 1, TC_COLS), jnp.int32),
        scratch_shapes=[
            pltpu.VMEM((1, TC_COLS), jnp.float32),
            pltpu.VMEM((1, TC_COLS), jnp.int32),
        ],
    )(x, x)


def _merge_body(ti_ref, pv_ref, pi_ref, o_ref):
    o_ref[:, pl.ds(0, TC_COLS)] = ti_ref[:, 0, :]
    mv = pv_ref[:, pl.ds(0, SC_COLS)]
    mi = pi_ref[:, pl.ds(0, SC_COLS)]
    for q in range(1, NQ):
        qv = pv_ref[:, pl.ds(q * SC_COLS, SC_COLS)]
        qi = pi_ref[:, pl.ds(q * SC_COLS, SC_COLS)]
        better = qv < mv
        mi = jnp.where(better, qi, mi)
        mv = jnp.where(better, qv, mv)
    o_ref[:, pl.ds(TC_COLS, SC_COLS)] = mi


def _merge(ti, pv, pi):
    return pl.pallas_call(
        _merge_body,
        out_shape=jax.ShapeDtypeStruct((B, L), jnp.int32),
    )(ti, pv, pi)


def kernel(x):
    pv, pi = _sc_argmin(x)
    ti = _tc_argmin(x)
    return _merge(ti, pv, pi)
